# baseline (speedup 1.0000x reference)
import functools

import jax
import jax.numpy as jnp
from jax import lax
from jax.experimental import pallas as pl
from jax.experimental.pallas import tpu as pltpu
from jax.experimental.pallas import tpu_sc as plsc

N = 10000
H = 128
A = 16
E = 160000
T = 3

_NC = 2
_NS = 16
_NW = _NC * _NS
_CH = 112
_EPT = (2 * E) // _NW
_CHUNKS = -(-_EPT // _CH)
_EPT_P = _CHUNKS * _CH
_NPAD = 10112
_RPS = _NPAD // _NS
_ACC_ROWS = _NPAD
assert _CHUNKS % 6 == 0 and _CHUNKS >= 12 and _EPT_P % 8 == 0

_BLK = 1000


def _sc_aggregate(ids, table):
    mesh = plsc.VectorSubcoreMesh(core_axis_name="c", subcore_axis_name="s")

    @functools.partial(
        pl.kernel,
        out_type=jax.ShapeDtypeStruct((_NC, _NPAD, H), jnp.float32),
        mesh=mesh,
        scratch_types=[
            pltpu.VMEM((6, 2, _CH), jnp.int32),
            pltpu.VMEM((3, _CH, H), jnp.float32),
            pltpu.VMEM_SHARED((_ACC_ROWS, H), jnp.float32),
        ] + [pltpu.SemaphoreType.DMA] * 15,
    )
    def agg(ids_hbm, table_hbm, out_hbm, ibuf, rows, acc, *sems):
        c = lax.axis_index("c")
        s = lax.axis_index("s")
        wid = c * _NS + s
        isem = sems[0:6]
        gsem = sems[6:9]
        g2sem = sems[9:12]
        ssem = sems[12:15]
        _HC = _CH // 2

        def fire_idx(j, ib):
            pltpu.async_copy(ids_hbm.at[wid].at[j], ibuf.at[ib], isem[ib])

        def wait_idx(ib):
            pltpu.make_async_copy(ids_hbm.at[wid].at[0], ibuf.at[ib],
                                  isem[ib]).wait()

        def fire_gather(ib, rb):
            pltpu.async_copy(
                table_hbm.at[ibuf.at[ib].at[0].at[pl.ds(0, _HC)]],
                rows.at[rb].at[pl.ds(0, _HC)], gsem[rb])
            pltpu.async_copy(
                table_hbm.at[ibuf.at[ib].at[0].at[pl.ds(_HC, _HC)]],
                rows.at[rb].at[pl.ds(_HC, _HC)], g2sem[rb])

        def wait_gather(ib, rb):
            pltpu.make_async_copy(
                table_hbm.at[ibuf.at[ib].at[0].at[pl.ds(0, _HC)]],
                rows.at[rb].at[pl.ds(0, _HC)], gsem[rb]).wait()
            pltpu.make_async_copy(
                table_hbm.at[ibuf.at[ib].at[0].at[pl.ds(_HC, _HC)]],
                rows.at[rb].at[pl.ds(_HC, _HC)], g2sem[rb]).wait()

        def fire_scatter(ib, rb):
            pltpu.async_copy(rows.at[rb], acc.at[ibuf.at[ib].at[1]],
                             ssem[rb], add=True)

        def wait_scatter(ib, rb):
            pltpu.make_async_copy(rows.at[rb], acc.at[ibuf.at[ib].at[1]],
                                  ssem[rb]).wait()

        def zrow(r, _):
            def zcol(k, _):
                rows[0, r, pl.ds(k * 16, 16)] = jnp.zeros((16,), jnp.float32)
                return 0
            return lax.fori_loop(0, H // 16, zcol, 0)
        lax.fori_loop(0, _CH, zrow, 0)
        base_r = s * _RPS
        nz = _RPS // _CH
        for k in range(nz):
            pltpu.sync_copy(rows.at[0], acc.at[pl.ds(base_r + k * _CH, _CH)])
        rem = _RPS - nz * _CH
        if rem:
            pltpu.sync_copy(rows.at[0].at[pl.ds(0, rem)],
                            acc.at[pl.ds(base_r + nz * _CH, rem)])
        plsc.subcore_barrier()

        pltpu.sync_copy(ids_hbm.at[wid].at[0], ibuf.at[0])
        for jj in range(1, 5):
            fire_idx(jj, jj)
        fire_gather(0, 0)
        wait_idx(1)
        fire_gather(1, 1)

        def step6(j, jm6, rb):
            rbp = (rb + 2) % 3
            ibp = (jm6 + 5) % 6
            ib2 = (jm6 + 2) % 6
            wait_gather(jm6, rb)
            fire_scatter(jm6, rb)
            @pl.when(j > 0)
            def _():
                wait_scatter(ibp, rbp)
            @pl.when(j + 5 < _CHUNKS)
            def _():
                fire_idx(j + 5, ibp)
            @pl.when(j + 2 < _CHUNKS)
            def _():
                wait_idx(ib2)
                fire_gather(ib2, rbp)

        def six(i, _):
            j0 = 6 * i
            for k in range(6):
                step6(j0 + k, k, k % 3)
            return 0
        lax.fori_loop(0, _CHUNKS // 6, six, 0)
        wait_scatter((_CHUNKS - 1) % 6, (_CHUNKS - 1) % 3)

        plsc.subcore_barrier()

        pltpu.sync_copy(acc.at[pl.ds(base_r, _RPS)],
                        out_hbm.at[c].at[pl.ds(base_r, _RPS)])

    return agg(ids, table)


def _emit_pre(h, wcat_ref, bcat_ref, tt_ref, gh_ref):
    out = jnp.dot(h, wcat_ref[...], preferred_element_type=jnp.float32)
    out = out + bcat_ref[...]
    tt_ref[0] = out[:, :H]
    tt_ref[1] = out[:, H:2 * H]
    gh_ref[...] = out[:, 2 * H:]


def _init_pre_body(x_ref, ann_ref, wt_ref, b_ref, wcat_ref, bcat_ref,
                   h_ref, tt_ref, gh_ref):
    h = (jnp.dot(x_ref[...], wt_ref[:H], preferred_element_type=jnp.float32)
         + jnp.dot(ann_ref[...], wt_ref[H:], preferred_element_type=jnp.float32)
         + b_ref[...])
    h_ref[...] = h
    _emit_pre(h, wcat_ref, bcat_ref, tt_ref, gh_ref)


def _tc_init_pre(x, ann, wh_t, bh, wcat, bcat):
    return pl.pallas_call(
        _init_pre_body,
        grid=(N // _BLK,),
        in_specs=[
            pl.BlockSpec((_BLK, H), lambda i: (i, 0)),
            pl.BlockSpec((_BLK, A), lambda i: (i, 0)),
            pl.BlockSpec((H + A, H), lambda i: (0, 0)),
            pl.BlockSpec((1, H), lambda i: (0, 0)),
            pl.BlockSpec((H, 5 * H), lambda i: (0, 0)),
            pl.BlockSpec((1, 5 * H), lambda i: (0, 0)),
        ],
        out_specs=[
            pl.BlockSpec((_BLK, H), lambda i: (i, 0)),
            pl.BlockSpec((2, _BLK, H), lambda i: (0, i, 0)),
            pl.BlockSpec((_BLK, 3 * H), lambda i: (i, 0)),
        ],
        out_shape=[
            jax.ShapeDtypeStruct((N, H), jnp.float32),
            jax.ShapeDtypeStruct((2, N, H), jnp.float32),
            jax.ShapeDtypeStruct((N, 3 * H), jnp.float32),
        ],
    )(x, ann, wh_t, bh, wcat, bcat)


def _gru_new_h(acc_ref, h_ref, gh_ref, wih_t_ref, bih_ref):
    inc = acc_ref[0] + acc_ref[1]
    gi = jnp.dot(inc, wih_t_ref[...], preferred_element_type=jnp.float32)
    gi = gi + bih_ref[...]
    gh = gh_ref[...]
    r = jax.nn.sigmoid(gi[:, :H] + gh[:, :H])
    z = jax.nn.sigmoid(gi[:, H:2 * H] + gh[:, H:2 * H])
    n = jnp.tanh(gi[:, 2 * H:] + r * gh[:, 2 * H:])
    return (1.0 - z) * n + z * h_ref[...]


def _step_body(acc_ref, h_ref, gh_ref, wih_t_ref, bih_ref, wcat_ref, bcat_ref,
               hnew_ref, tt_ref, ghn_ref):
    hn = _gru_new_h(acc_ref, h_ref, gh_ref, wih_t_ref, bih_ref)
    hnew_ref[...] = hn
    _emit_pre(hn, wcat_ref, bcat_ref, tt_ref, ghn_ref)


def _tc_step(acc, h, gh, wih_t, bih, wcat, bcat):
    return pl.pallas_call(
        _step_body,
        grid=(N // _BLK,),
        in_specs=[
            pl.BlockSpec((2, _BLK, H), lambda i: (0, i, 0)),
            pl.BlockSpec((_BLK, H), lambda i: (i, 0)),
            pl.BlockSpec((_BLK, 3 * H), lambda i: (i, 0)),
            pl.BlockSpec((H, 3 * H), lambda i: (0, 0)),
            pl.BlockSpec((1, 3 * H), lambda i: (0, 0)),
            pl.BlockSpec((H, 5 * H), lambda i: (0, 0)),
            pl.BlockSpec((1, 5 * H), lambda i: (0, 0)),
        ],
        out_specs=[
            pl.BlockSpec((_BLK, H), lambda i: (i, 0)),
            pl.BlockSpec((2, _BLK, H), lambda i: (0, i, 0)),
            pl.BlockSpec((_BLK, 3 * H), lambda i: (i, 0)),
        ],
        out_shape=[
            jax.ShapeDtypeStruct((N, H), jnp.float32),
            jax.ShapeDtypeStruct((2, N, H), jnp.float32),
            jax.ShapeDtypeStruct((N, 3 * H), jnp.float32),
        ],
    )(acc, h, gh, wih_t, bih, wcat, bcat)


def _post_body(acc_ref, h_ref, gh_ref, wih_t_ref, bih_ref, hnew_ref):
    hnew_ref[...] = _gru_new_h(acc_ref, h_ref, gh_ref, wih_t_ref, bih_ref)


def _tc_post(acc, h, gh, wih_t, bih):
    return pl.pallas_call(
        _post_body,
        grid=(N // _BLK,),
        in_specs=[
            pl.BlockSpec((2, _BLK, H), lambda i: (0, i, 0)),
            pl.BlockSpec((_BLK, H), lambda i: (i, 0)),
            pl.BlockSpec((_BLK, 3 * H), lambda i: (i, 0)),
            pl.BlockSpec((H, 3 * H), lambda i: (0, 0)),
            pl.BlockSpec((1, 3 * H), lambda i: (0, 0)),
        ],
        out_specs=pl.BlockSpec((_BLK, H), lambda i: (i, 0)),
        out_shape=jax.ShapeDtypeStruct((N, H), jnp.float32),
    )(acc, h, gh, wih_t, bih)


def kernel(initial_node_representation, annotations, adj0, adj1, W_hidden,
           b_hidden, W_msg0, b_msg0, W_msg1, b_msg1, W_ih, W_hh, b_ih, b_hh):
    wh_t = W_hidden.T
    wcat = jnp.concatenate([W_msg0.T, W_msg1.T, W_hh.T], axis=1)
    bcat = jnp.concatenate([b_msg0, b_msg1, b_hh])[None]
    wih_t = W_ih.T

    src = jnp.concatenate([adj0[:, 0], adj1[:, 0] + N])
    dst = jnp.concatenate([adj0[:, 1], adj1[:, 1]])
    per = (2 * E) // _NW
    src_p = jnp.pad(src.reshape(_NW, per), ((0, 0), (0, _EPT_P - per)),
                    constant_values=0).reshape(_NW, _CHUNKS, _CH)
    dst_pad = N + (jnp.arange(_EPT_P - per, dtype=jnp.int32) % (_NPAD - N))
    dst_p = jnp.concatenate(
        [dst.reshape(_NW, per),
         jnp.broadcast_to(dst_pad, (_NW, _EPT_P - per))],
        axis=1).reshape(_NW, _CHUNKS, _CH)
    ids = jnp.stack([src_p, dst_p], axis=2)

    h, tt, gh = _tc_init_pre(initial_node_representation, annotations, wh_t,
                             b_hidden[None], wcat, bcat)
    for t in range(T):
        acc = _sc_aggregate(ids, tt.reshape(2 * N, H))
        if t < T - 1:
            h, tt, gh = _tc_step(acc, h, gh, wih_t, b_ih[None], wcat, bcat)
        else:
            h = _tc_post(acc, h, gh, wih_t, b_ih[None])
    return h


# speedup vs baseline: 1.0103x; 1.0103x over previous
import functools

import jax
import jax.numpy as jnp
from jax import lax
from jax.experimental import pallas as pl
from jax.experimental.pallas import tpu as pltpu
from jax.experimental.pallas import tpu_sc as plsc

N = 10000
H = 128
A = 16
E = 160000
T = 3

_NC = 2
_NS = 16
_NW = _NC * _NS
_CH = 112
_EPT = (2 * E) // _NW
_CHUNKS = -(-_EPT // _CH)
_EPT_P = _CHUNKS * _CH
_NPAD = 10112
_RPS = _NPAD // _NS
_ACC_ROWS = _NPAD
assert _CHUNKS % 6 == 0 and _CHUNKS >= 12 and _EPT_P % 8 == 0

_BLK = 1000


def _sc_aggregate(ids, table):
    mesh = plsc.VectorSubcoreMesh(core_axis_name="c", subcore_axis_name="s")

    @functools.partial(
        pl.kernel,
        out_type=jax.ShapeDtypeStruct((_NC, _NPAD, H), jnp.float32),
        mesh=mesh,
        scratch_types=[
            pltpu.VMEM((6, 2, _CH), jnp.int32),
            pltpu.VMEM((3, _CH, H), jnp.float32),
            pltpu.VMEM_SHARED((_ACC_ROWS, H), jnp.float32),
        ] + [pltpu.SemaphoreType.DMA] * 12,
    )
    def agg(ids_hbm, table_hbm, out_hbm, ibuf, rows, acc, *sems):
        c = lax.axis_index("c")
        s = lax.axis_index("s")
        wid = c * _NS + s
        isem = sems[0:6]
        gsem = sems[6:9]
        ssem = sems[9:12]

        def fire_idx(j, ib):
            pltpu.async_copy(ids_hbm.at[wid].at[j], ibuf.at[ib], isem[ib])

        def wait_idx(ib):
            pltpu.make_async_copy(ids_hbm.at[wid].at[0], ibuf.at[ib],
                                  isem[ib]).wait()

        def fire_gather(ib, rb):
            pltpu.async_copy(table_hbm.at[ibuf.at[ib].at[0]], rows.at[rb],
                             gsem[rb])

        def wait_gather(ib, rb):
            pltpu.make_async_copy(table_hbm.at[ibuf.at[ib].at[0]],
                                  rows.at[rb], gsem[rb]).wait()

        def fire_scatter(ib, rb):
            pltpu.async_copy(rows.at[rb], acc.at[ibuf.at[ib].at[1]],
                             ssem[rb], add=True)

        def wait_scatter(ib, rb):
            pltpu.make_async_copy(rows.at[rb], acc.at[ibuf.at[ib].at[1]],
                                  ssem[rb]).wait()

        def zrow(r, _):
            def zcol(k, _):
                rows[0, r, pl.ds(k * 16, 16)] = jnp.zeros((16,), jnp.float32)
                return 0
            return lax.fori_loop(0, H // 16, zcol, 0)
        lax.fori_loop(0, _CH, zrow, 0)
        base_r = s * _RPS
        nz = _RPS // _CH
        for k in range(nz):
            pltpu.sync_copy(rows.at[0], acc.at[pl.ds(base_r + k * _CH, _CH)])
        rem = _RPS - nz * _CH
        if rem:
            pltpu.sync_copy(rows.at[0].at[pl.ds(0, rem)],
                            acc.at[pl.ds(base_r + nz * _CH, rem)])
        plsc.subcore_barrier()

        pltpu.sync_copy(ids_hbm.at[wid].at[0], ibuf.at[0])
        for jj in range(1, 5):
            fire_idx(jj, jj)
        fire_gather(0, 0)
        wait_idx(1)
        fire_gather(1, 1)

        def step6(j, jm6, rb):
            rbp = (rb + 2) % 3
            ibp = (jm6 + 5) % 6
            ib2 = (jm6 + 2) % 6
            wait_gather(jm6, rb)
            fire_scatter(jm6, rb)
            @pl.when(j > 0)
            def _():
                wait_scatter(ibp, rbp)
            @pl.when(j + 5 < _CHUNKS)
            def _():
                fire_idx(j + 5, ibp)
            @pl.when(j + 2 < _CHUNKS)
            def _():
                wait_idx(ib2)
                fire_gather(ib2, rbp)

        def six(i, _):
            j0 = 6 * i
            for k in range(6):
                step6(j0 + k, k, k % 3)
            return 0
        lax.fori_loop(0, _CHUNKS // 6, six, 0)
        wait_scatter((_CHUNKS - 1) % 6, (_CHUNKS - 1) % 3)

        plsc.subcore_barrier()

        pltpu.sync_copy(acc.at[pl.ds(base_r, _RPS)],
                        out_hbm.at[c].at[pl.ds(base_r, _RPS)])

    return agg(ids, table)


def _emit_pre(h, wcat_ref, bcat_ref, tt_ref, gh_ref):
    out = jnp.dot(h, wcat_ref[...], preferred_element_type=jnp.float32)
    out = out + bcat_ref[...]
    tt_ref[0] = out[:, :H]
    tt_ref[1] = out[:, H:2 * H]
    gh_ref[...] = out[:, 2 * H:]


def _init_pre_body(x_ref, ann_ref, wt_ref, b_ref, wcat_ref, bcat_ref,
                   h_ref, tt_ref, gh_ref):
    h = (jnp.dot(x_ref[...], wt_ref[:H], preferred_element_type=jnp.float32)
         + jnp.dot(ann_ref[...], wt_ref[H:], preferred_element_type=jnp.float32)
         + b_ref[...])
    h_ref[...] = h
    _emit_pre(h, wcat_ref, bcat_ref, tt_ref, gh_ref)


def _tc_init_pre(x, ann, wh_t, bh, wcat, bcat):
    return pl.pallas_call(
        _init_pre_body,
        grid=(N // _BLK,),
        in_specs=[
            pl.BlockSpec((_BLK, H), lambda i: (i, 0)),
            pl.BlockSpec((_BLK, A), lambda i: (i, 0)),
            pl.BlockSpec((H + A, H), lambda i: (0, 0)),
            pl.BlockSpec((1, H), lambda i: (0, 0)),
            pl.BlockSpec((H, 5 * H), lambda i: (0, 0)),
            pl.BlockSpec((1, 5 * H), lambda i: (0, 0)),
        ],
        out_specs=[
            pl.BlockSpec((_BLK, H), lambda i: (i, 0)),
            pl.BlockSpec((2, _BLK, H), lambda i: (0, i, 0)),
            pl.BlockSpec((_BLK, 3 * H), lambda i: (i, 0)),
        ],
        out_shape=[
            jax.ShapeDtypeStruct((N, H), jnp.float32),
            jax.ShapeDtypeStruct((2, N, H), jnp.float32),
            jax.ShapeDtypeStruct((N, 3 * H), jnp.float32),
        ],
    )(x, ann, wh_t, bh, wcat, bcat)


def _gru_new_h(acc_ref, h_ref, gh_ref, wih_t_ref, bih_ref):
    inc = acc_ref[0] + acc_ref[1]
    gi = jnp.dot(inc, wih_t_ref[...], preferred_element_type=jnp.float32)
    gi = gi + bih_ref[...]
    gh = gh_ref[...]
    r = jax.nn.sigmoid(gi[:, :H] + gh[:, :H])
    z = jax.nn.sigmoid(gi[:, H:2 * H] + gh[:, H:2 * H])
    n = jnp.tanh(gi[:, 2 * H:] + r * gh[:, 2 * H:])
    return (1.0 - z) * n + z * h_ref[...]


def _step_body(acc_ref, h_ref, gh_ref, wih_t_ref, bih_ref, wcat_ref, bcat_ref,
               hnew_ref, tt_ref, ghn_ref):
    hn = _gru_new_h(acc_ref, h_ref, gh_ref, wih_t_ref, bih_ref)
    hnew_ref[...] = hn
    _emit_pre(hn, wcat_ref, bcat_ref, tt_ref, ghn_ref)


def _tc_step(acc, h, gh, wih_t, bih, wcat, bcat):
    return pl.pallas_call(
        _step_body,
        grid=(N // _BLK,),
        in_specs=[
            pl.BlockSpec((2, _BLK, H), lambda i: (0, i, 0)),
            pl.BlockSpec((_BLK, H), lambda i: (i, 0)),
            pl.BlockSpec((_BLK, 3 * H), lambda i: (i, 0)),
            pl.BlockSpec((H, 3 * H), lambda i: (0, 0)),
            pl.BlockSpec((1, 3 * H), lambda i: (0, 0)),
            pl.BlockSpec((H, 5 * H), lambda i: (0, 0)),
            pl.BlockSpec((1, 5 * H), lambda i: (0, 0)),
        ],
        out_specs=[
            pl.BlockSpec((_BLK, H), lambda i: (i, 0)),
            pl.BlockSpec((2, _BLK, H), lambda i: (0, i, 0)),
            pl.BlockSpec((_BLK, 3 * H), lambda i: (i, 0)),
        ],
        out_shape=[
            jax.ShapeDtypeStruct((N, H), jnp.float32),
            jax.ShapeDtypeStruct((2, N, H), jnp.float32),
            jax.ShapeDtypeStruct((N, 3 * H), jnp.float32),
        ],
    )(acc, h, gh, wih_t, bih, wcat, bcat)


def _post_body(acc_ref, h_ref, gh_ref, wih_t_ref, bih_ref, hnew_ref):
    hnew_ref[...] = _gru_new_h(acc_ref, h_ref, gh_ref, wih_t_ref, bih_ref)


def _tc_post(acc, h, gh, wih_t, bih):
    return pl.pallas_call(
        _post_body,
        grid=(N // _BLK,),
        in_specs=[
            pl.BlockSpec((2, _BLK, H), lambda i: (0, i, 0)),
            pl.BlockSpec((_BLK, H), lambda i: (i, 0)),
            pl.BlockSpec((_BLK, 3 * H), lambda i: (i, 0)),
            pl.BlockSpec((H, 3 * H), lambda i: (0, 0)),
            pl.BlockSpec((1, 3 * H), lambda i: (0, 0)),
        ],
        out_specs=pl.BlockSpec((_BLK, H), lambda i: (i, 0)),
        out_shape=jax.ShapeDtypeStruct((N, H), jnp.float32),
    )(acc, h, gh, wih_t, bih)


def kernel(initial_node_representation, annotations, adj0, adj1, W_hidden,
           b_hidden, W_msg0, b_msg0, W_msg1, b_msg1, W_ih, W_hh, b_ih, b_hh):
    wh_t = W_hidden.T
    wcat = jnp.concatenate([W_msg0.T, W_msg1.T, W_hh.T], axis=1)
    bcat = jnp.concatenate([b_msg0, b_msg1, b_hh])[None]
    wih_t = W_ih.T

    src = jnp.concatenate([adj0[:, 0], adj1[:, 0] + N])
    dst = jnp.concatenate([adj0[:, 1], adj1[:, 1]])
    per = (2 * E) // _NW
    src_p = jnp.pad(src.reshape(_NW, per), ((0, 0), (0, _EPT_P - per)),
                    constant_values=0).reshape(_NW, _CHUNKS, _CH)
    dst_pad = N + (jnp.arange(_EPT_P - per, dtype=jnp.int32) % (_NPAD - N))
    dst_p = jnp.concatenate(
        [dst.reshape(_NW, per),
         jnp.broadcast_to(dst_pad, (_NW, _EPT_P - per))],
        axis=1).reshape(_NW, _CHUNKS, _CH)
    ids = jnp.stack([src_p, dst_p], axis=2)

    h, tt, gh = _tc_init_pre(initial_node_representation, annotations, wh_t,
                             b_hidden[None], wcat, bcat)
    for t in range(T):
        acc = _sc_aggregate(ids, tt.reshape(2 * N, H))
        if t < T - 1:
            h, tt, gh = _tc_step(acc, h, gh, wih_t, b_ih[None], wcat, bcat)
        else:
            h = _tc_post(acc, h, gh, wih_t, b_ih[None])
    return h


# speedup vs baseline: 1.0109x; 1.0006x over previous
import functools

import jax
import jax.numpy as jnp
from jax import lax
from jax.experimental import pallas as pl
from jax.experimental.pallas import tpu as pltpu
from jax.experimental.pallas import tpu_sc as plsc

N = 10000
H = 128
A = 16
E = 160000
T = 3

_NC = 2
_NS = 16
_NW = _NC * _NS
_CH = 112
_EPT = (2 * E) // _NW
_CHUNKS = -(-_EPT // _CH)
_EPT_P = _CHUNKS * _CH
_NPAD = 10112
_RPS = _NPAD // _NS
_ACC_ROWS = _NPAD
assert _CHUNKS % 6 == 0 and _CHUNKS >= 12 and _EPT_P % 8 == 0

_BLK = 1000


def _sc_aggregate(ids, table):
    mesh = plsc.VectorSubcoreMesh(core_axis_name="c", subcore_axis_name="s")

    @functools.partial(
        pl.kernel,
        out_type=jax.ShapeDtypeStruct((_NC, _NPAD, H), jnp.float32),
        mesh=mesh,
        scratch_types=[
            pltpu.VMEM((6, 2, _CH), jnp.int32),
            pltpu.VMEM((3, _CH, H), jnp.float32),
            pltpu.VMEM_SHARED((_ACC_ROWS, H), jnp.float32),
        ] + [pltpu.SemaphoreType.DMA] * 12,
    )
    def agg(ids_hbm, table_hbm, out_hbm, ibuf, rows, acc, *sems):
        c = lax.axis_index("c")
        s = lax.axis_index("s")
        wid = c * _NS + s
        isem = sems[0:6]
        gsem = sems[6:9]
        ssem = sems[9:12]

        def fire_idx(j, ib):
            pltpu.async_copy(ids_hbm.at[wid].at[j], ibuf.at[ib], isem[ib])

        def wait_idx(ib):
            pltpu.make_async_copy(ids_hbm.at[wid].at[0], ibuf.at[ib],
                                  isem[ib]).wait()

        def fire_gather(ib, rb):
            pltpu.async_copy(table_hbm.at[ibuf.at[ib].at[0]], rows.at[rb],
                             gsem[rb])

        def wait_gather(ib, rb):
            pltpu.make_async_copy(table_hbm.at[ibuf.at[ib].at[0]],
                                  rows.at[rb], gsem[rb]).wait()

        def fire_scatter(ib, rb):
            pltpu.async_copy(rows.at[rb], acc.at[ibuf.at[ib].at[1]],
                             ssem[rb], add=True)

        def wait_scatter(ib, rb):
            pltpu.make_async_copy(rows.at[rb], acc.at[ibuf.at[ib].at[1]],
                                  ssem[rb]).wait()

        def zrow(r, _):
            def zcol(k, _):
                rows[0, r, pl.ds(k * 16, 16)] = jnp.zeros((16,), jnp.float32)
                return 0
            return lax.fori_loop(0, H // 16, zcol, 0)
        lax.fori_loop(0, _CH, zrow, 0)
        base_r = s * _RPS
        nz = _RPS // _CH
        for k in range(nz):
            pltpu.sync_copy(rows.at[0], acc.at[pl.ds(base_r + k * _CH, _CH)])
        rem = _RPS - nz * _CH
        if rem:
            pltpu.sync_copy(rows.at[0].at[pl.ds(0, rem)],
                            acc.at[pl.ds(base_r + nz * _CH, rem)])
        plsc.subcore_barrier()

        pltpu.sync_copy(ids_hbm.at[wid].at[0], ibuf.at[0])
        for jj in range(1, 5):
            fire_idx(jj, jj)
        fire_gather(0, 0)
        wait_idx(1)
        fire_gather(1, 1)

        def step6(j, jm6, rb):
            rbp = (rb + 2) % 3
            ibp = (jm6 + 5) % 6
            ib2 = (jm6 + 2) % 6
            wait_gather(jm6, rb)
            fire_scatter(jm6, rb)
            @pl.when(j > 0)
            def _():
                wait_scatter(ibp, rbp)
            @pl.when(j + 5 < _CHUNKS)
            def _():
                fire_idx(j + 5, ibp)
            @pl.when(j + 2 < _CHUNKS)
            def _():
                wait_idx(ib2)
                fire_gather(ib2, rbp)

        def six(i, _):
            j0 = 6 * i
            for k in range(6):
                step6(j0 + k, k, k % 3)
            return 0
        lax.fori_loop(0, _CHUNKS // 6, six, 0)
        wait_scatter((_CHUNKS - 1) % 6, (_CHUNKS - 1) % 3)

        plsc.subcore_barrier()

        pltpu.sync_copy(acc.at[pl.ds(base_r, _RPS)],
                        out_hbm.at[c].at[pl.ds(base_r, _RPS)])

    return agg(ids, table)


def _emit_pre(h, wcat_ref, bcat_ref, tt_ref, gh_ref):
    out = jnp.dot(h.astype(jnp.bfloat16), wcat_ref[...],
                  preferred_element_type=jnp.float32)
    out = out + bcat_ref[...]
    tt_ref[0] = out[:, :H]
    tt_ref[1] = out[:, H:2 * H]
    gh_ref[...] = out[:, 2 * H:]


def _init_pre_body(x_ref, ann_ref, wt_ref, b_ref, wcat_ref, bcat_ref,
                   h_ref, tt_ref, gh_ref):
    h = (jnp.dot(x_ref[...].astype(jnp.bfloat16), wt_ref[:H],
                 preferred_element_type=jnp.float32)
         + jnp.dot(ann_ref[...].astype(jnp.bfloat16), wt_ref[H:],
                   preferred_element_type=jnp.float32)
         + b_ref[...])
    h_ref[...] = h
    _emit_pre(h, wcat_ref, bcat_ref, tt_ref, gh_ref)


def _tc_init_pre(x, ann, wh_t, bh, wcat, bcat):
    return pl.pallas_call(
        _init_pre_body,
        grid=(N // _BLK,),
        in_specs=[
            pl.BlockSpec((_BLK, H), lambda i: (i, 0)),
            pl.BlockSpec((_BLK, A), lambda i: (i, 0)),
            pl.BlockSpec((H + A, H), lambda i: (0, 0)),
            pl.BlockSpec((1, H), lambda i: (0, 0)),
            pl.BlockSpec((H, 5 * H), lambda i: (0, 0)),
            pl.BlockSpec((1, 5 * H), lambda i: (0, 0)),
        ],
        out_specs=[
            pl.BlockSpec((_BLK, H), lambda i: (i, 0)),
            pl.BlockSpec((2, _BLK, H), lambda i: (0, i, 0)),
            pl.BlockSpec((_BLK, 3 * H), lambda i: (i, 0)),
        ],
        out_shape=[
            jax.ShapeDtypeStruct((N, H), jnp.float32),
            jax.ShapeDtypeStruct((2, N, H), jnp.float32),
            jax.ShapeDtypeStruct((N, 3 * H), jnp.float32),
        ],
    )(x, ann, wh_t, bh, wcat, bcat)


def _gru_new_h(acc_ref, h_ref, gh_ref, wih_t_ref, bih_ref):
    inc = acc_ref[0] + acc_ref[1]
    gi = jnp.dot(inc.astype(jnp.bfloat16), wih_t_ref[...],
                 preferred_element_type=jnp.float32)
    gi = gi + bih_ref[...]
    gh = gh_ref[...]
    r = jax.nn.sigmoid(gi[:, :H] + gh[:, :H])
    z = jax.nn.sigmoid(gi[:, H:2 * H] + gh[:, H:2 * H])
    n = jnp.tanh(gi[:, 2 * H:] + r * gh[:, 2 * H:])
    return (1.0 - z) * n + z * h_ref[...]


def _step_body(acc_ref, h_ref, gh_ref, wih_t_ref, bih_ref, wcat_ref, bcat_ref,
               hnew_ref, tt_ref, ghn_ref):
    hn = _gru_new_h(acc_ref, h_ref, gh_ref, wih_t_ref, bih_ref)
    hnew_ref[...] = hn
    _emit_pre(hn, wcat_ref, bcat_ref, tt_ref, ghn_ref)


def _tc_step(acc, h, gh, wih_t, bih, wcat, bcat):
    return pl.pallas_call(
        _step_body,
        grid=(N // _BLK,),
        in_specs=[
            pl.BlockSpec((2, _BLK, H), lambda i: (0, i, 0)),
            pl.BlockSpec((_BLK, H), lambda i: (i, 0)),
            pl.BlockSpec((_BLK, 3 * H), lambda i: (i, 0)),
            pl.BlockSpec((H, 3 * H), lambda i: (0, 0)),
            pl.BlockSpec((1, 3 * H), lambda i: (0, 0)),
            pl.BlockSpec((H, 5 * H), lambda i: (0, 0)),
            pl.BlockSpec((1, 5 * H), lambda i: (0, 0)),
        ],
        out_specs=[
            pl.BlockSpec((_BLK, H), lambda i: (i, 0)),
            pl.BlockSpec((2, _BLK, H), lambda i: (0, i, 0)),
            pl.BlockSpec((_BLK, 3 * H), lambda i: (i, 0)),
        ],
        out_shape=[
            jax.ShapeDtypeStruct((N, H), jnp.float32),
            jax.ShapeDtypeStruct((2, N, H), jnp.float32),
            jax.ShapeDtypeStruct((N, 3 * H), jnp.float32),
        ],
    )(acc, h, gh, wih_t, bih, wcat, bcat)


def _post_body(acc_ref, h_ref, gh_ref, wih_t_ref, bih_ref, hnew_ref):
    hnew_ref[...] = _gru_new_h(acc_ref, h_ref, gh_ref, wih_t_ref, bih_ref)


def _tc_post(acc, h, gh, wih_t, bih):
    return pl.pallas_call(
        _post_body,
        grid=(N // _BLK,),
        in_specs=[
            pl.BlockSpec((2, _BLK, H), lambda i: (0, i, 0)),
            pl.BlockSpec((_BLK, H), lambda i: (i, 0)),
            pl.BlockSpec((_BLK, 3 * H), lambda i: (i, 0)),
            pl.BlockSpec((H, 3 * H), lambda i: (0, 0)),
            pl.BlockSpec((1, 3 * H), lambda i: (0, 0)),
        ],
        out_specs=pl.BlockSpec((_BLK, H), lambda i: (i, 0)),
        out_shape=jax.ShapeDtypeStruct((N, H), jnp.float32),
    )(acc, h, gh, wih_t, bih)


def kernel(initial_node_representation, annotations, adj0, adj1, W_hidden,
           b_hidden, W_msg0, b_msg0, W_msg1, b_msg1, W_ih, W_hh, b_ih, b_hh):
    wh_t = W_hidden.T.astype(jnp.bfloat16)
    wcat = jnp.concatenate([W_msg0.T, W_msg1.T, W_hh.T],
                           axis=1).astype(jnp.bfloat16)
    bcat = jnp.concatenate([b_msg0, b_msg1, b_hh])[None]
    wih_t = W_ih.T.astype(jnp.bfloat16)

    src = jnp.concatenate([adj0[:, 0], adj1[:, 0] + N])
    dst = jnp.concatenate([adj0[:, 1], adj1[:, 1]])
    per = (2 * E) // _NW
    src_p = jnp.pad(src.reshape(_NW, per), ((0, 0), (0, _EPT_P - per)),
                    constant_values=0).reshape(_NW, _CHUNKS, _CH)
    dst_pad = N + (jnp.arange(_EPT_P - per, dtype=jnp.int32) % (_NPAD - N))
    dst_p = jnp.concatenate(
        [dst.reshape(_NW, per),
         jnp.broadcast_to(dst_pad, (_NW, _EPT_P - per))],
        axis=1).reshape(_NW, _CHUNKS, _CH)
    ids = jnp.stack([src_p, dst_p], axis=2)

    h, tt, gh = _tc_init_pre(initial_node_representation, annotations, wh_t,
                             b_hidden[None], wcat, bcat)
    for t in range(T):
        acc = _sc_aggregate(ids, tt.reshape(2 * N, H))
        if t < T - 1:
            h, tt, gh = _tc_step(acc, h, gh, wih_t, b_ih[None], wcat, bcat)
        else:
            h = _tc_post(acc, h, gh, wih_t, b_ih[None])
    return h
